# Initial kernel scaffold; baseline (speedup 1.0000x reference)
import functools

import jax
import jax.numpy as jnp
from jax import lax
from jax.experimental import pallas as pl
from jax.experimental.pallas import tpu as pltpu
from jax.experimental.pallas import tpu_sc as plsc

K = 16
N_PAD = 10240
ROWS = 256
CHUNK = 512
N_TILES = N_PAD // ROWS
N_CHUNKS = N_PAD // CHUNK
BIG_I = 2 ** 30
F32_INF = jnp.float32(jnp.inf)


def _knn_body(ids_ref, novl_ref, q_ref, c3_ref, idx_out_ref, d2_ref):
    t = pl.program_id(0)
    nov = novl_ref[t]
    qx = q_ref[:, 0:1]
    qy = q_ref[:, 1:2]
    qz = q_ref[:, 2:3]
    qb = q_ref[:, 3:4]

    def fill(k, carry):
        cid = ids_ref[t, k]
        cx = c3_ref[cid, 0:1, :]
        cy = c3_ref[cid, 1:2, :]
        cz = c3_ref[cid, 2:3, :]
        cb = c3_ref[cid, 3:4, :]
        dx = qx - cx
        dy = qy - cy
        dz = qz - cz
        d2 = dx * dx + dy * dy + dz * dz
        d2_ref[cid] = jnp.where(qb == cb, d2, F32_INF)
        return carry

    lax.fori_loop(0, nov, fill, 0)

    for tsel in range(K):
        def sel(k, mi):
            m, i = mi
            cid = ids_ref[t, k]
            d2c = d2_ref[cid]
            mc = jnp.min(d2c, axis=1, keepdims=True)
            colidx = (lax.broadcasted_iota(jnp.int32, (ROWS, CHUNK), 1)
                      + cid * CHUNK)
            ic = jnp.min(jnp.where(d2c == mc, colidx, BIG_I), axis=1,
                         keepdims=True)
            better = (mc < m) | ((mc == m) & (ic < i))
            return (jnp.where(better, mc, m), jnp.where(better, ic, i))

        m0 = jnp.full((ROWS, 1), F32_INF, jnp.float32)
        i0 = jnp.full((ROWS, 1), BIG_I, jnp.int32)
        m, i = lax.fori_loop(0, nov, sel, (m0, i0))

        def knock(k, carry):
            cid = ids_ref[t, k]
            d2c = d2_ref[cid]
            colidx = (lax.broadcasted_iota(jnp.int32, (ROWS, CHUNK), 1)
                      + cid * CHUNK)
            d2_ref[cid] = jnp.where((d2c == m) & (colidx == i), F32_INF, d2c)
            return carry

        lax.fori_loop(0, nov, knock, 0)
        safe = jnp.minimum(i, N_PAD - 1)
        idx_out_ref[:, tsel:tsel + 1] = jnp.where(qb == -2.0, N_PAD - 1, safe)


def _knn_topk(Q, C3, ids, novl, interpret=False):
    grid_spec = pltpu.PrefetchScalarGridSpec(
        num_scalar_prefetch=2,
        grid=(N_TILES,),
        in_specs=[
            pl.BlockSpec((ROWS, 4), lambda t, *_: (t, 0)),
            pl.BlockSpec((N_CHUNKS, 8, CHUNK), lambda t, *_: (0, 0, 0)),
        ],
        out_specs=pl.BlockSpec((ROWS, K), lambda t, *_: (t, 0)),
        scratch_shapes=[pltpu.VMEM((N_CHUNKS, ROWS, CHUNK), jnp.float32)],
    )
    return pl.pallas_call(
        _knn_body,
        grid_spec=grid_spec,
        out_shape=jax.ShapeDtypeStruct((N_PAD, K), jnp.int32),
        interpret=interpret,
    )(ids, novl, Q, C3)


def _scatter_variance(x_sc, idx_flat, zeros_slab):
    C = x_sc.shape[1]
    slab = N_PAD // 16
    qpt = N_PAD // 32
    n_blocks = qpt // 8
    mesh = plsc.VectorSubcoreMesh(core_axis_name="c", subcore_axis_name="s")

    @functools.partial(
        pl.kernel,
        mesh=mesh,
        out_type=jax.ShapeDtypeStruct((2, N_PAD, C), jnp.float32),
        scratch_types=[
            pltpu.VMEM((128,), jnp.int32),
            pltpu.VMEM((8, C), jnp.float32),
            pltpu.VMEM((128, C), jnp.float32),
            pltpu.VMEM_SHARED((N_PAD, C), jnp.float32),
            pltpu.SemaphoreType.DMA,
        ],
    )
    def sc_kernel(x_hbm, idxf_hbm, z_hbm, out_hbm, idx_v, cent_v, nbr_v,
                  var_sh, sem):
        c = lax.axis_index("c")
        s = lax.axis_index("s")
        base = (c * 16 + s) * qpt
        pltpu.sync_copy(z_hbm, var_sh.at[pl.ds(s * slab, slab)])
        plsc.subcore_barrier()

        def block(b, carry):
            q0 = base + b * 8
            pltpu.sync_copy(idxf_hbm.at[pl.ds(q0 * K, 128)], idx_v)
            pltpu.sync_copy(x_hbm.at[pl.ds(q0, 8)], cent_v)
            pltpu.async_copy(x_hbm.at[idx_v], nbr_v, sem).wait()

            def rowloop(r, carry2):
                q = lax.shift_right_logical(r, 4)
                for cc in range(C // 16):
                    sl = pl.ds(cc * 16, 16)
                    d = nbr_v[r, sl] - cent_v[q, sl]
                    nbr_v[r, sl] = d * d
                return carry2

            lax.fori_loop(0, 128, rowloop, 0)
            pltpu.sync_copy(nbr_v, var_sh.at[idx_v], add=True)
            return carry

        lax.fori_loop(0, n_blocks, block, 0)
        plsc.subcore_barrier()
        pltpu.sync_copy(var_sh.at[pl.ds(s * slab, slab)],
                        out_hbm.at[c, pl.ds(s * slab, slab)])

    return sc_kernel(x_sc, idx_flat, zeros_slab)


def _mlp_body(v0_ref, v1_ref, w1t_ref, b1_ref, w2_ref, b2_ref, out_ref):
    v = (v0_ref[...] + v1_ref[...]) * jnp.float32(1.0 / K)
    h = jnp.dot(v, w1t_ref[...], preferred_element_type=jnp.float32)
    h = jnp.maximum(h + b1_ref[...], 0.0)
    s = jnp.sum(h * w2_ref[...], axis=1, keepdims=True) + b2_ref[...]
    out_ref[...] = 1.0 / (1.0 + jnp.exp(-s))


def _mlp(var0, var1, W1t, b1r, W2r, b2r, interpret=False):
    RT = 512
    grid = (N_PAD // RT,)
    return pl.pallas_call(
        _mlp_body,
        grid=grid,
        in_specs=[
            pl.BlockSpec((RT, 128), lambda r: (r, 0)),
            pl.BlockSpec((RT, 128), lambda r: (r, 0)),
            pl.BlockSpec((128, 64), lambda r: (0, 0)),
            pl.BlockSpec((1, 64), lambda r: (0, 0)),
            pl.BlockSpec((1, 64), lambda r: (0, 0)),
            pl.BlockSpec((1, 1), lambda r: (0, 0)),
        ],
        out_specs=pl.BlockSpec((RT, 1), lambda r: (r, 0)),
        out_shape=jax.ShapeDtypeStruct((N_PAD, 1), jnp.float32),
        interpret=interpret,
    )(var0, var1, W1t, b1r, W2r, b2r)


def _prep_knn_inputs(pos, batch):
    N = pos.shape[0]
    bf = batch.astype(jnp.float32)
    Q = jnp.full((N_PAD, 4), -2.0, jnp.float32)
    Q = Q.at[:N, :3].set(pos)
    Q = Q.at[:N, 3].set(bf)
    Carr = jnp.zeros((8, N_PAD), jnp.float32)
    Carr = Carr.at[:3, :N].set(pos.T)
    Carr = Carr.at[3, :].set(-1.0)
    Carr = Carr.at[3, :N].set(bf)
    C3 = Carr.reshape(8, N_CHUNKS, CHUNK).transpose(1, 0, 2)
    tix = jnp.arange(N_TILES)
    cix = jnp.arange(N_CHUNKS)
    tmin = batch[jnp.minimum(tix * ROWS, N - 1)]
    tmax = batch[jnp.minimum((tix + 1) * ROWS - 1, N - 1)]
    cmin = batch[jnp.minimum(cix * CHUNK, N - 1)]
    cmax = batch[jnp.minimum((cix + 1) * CHUNK - 1, N - 1)]
    ovl = (cmax[None, :] >= tmin[:, None]) & (cmin[None, :] <= tmax[:, None])
    novl = jnp.sum(ovl.astype(jnp.int32), axis=1)
    ids = jnp.argsort(
        jnp.where(ovl, cix[None, :], N_CHUNKS + 100), axis=1
    ).astype(jnp.int32)
    return Q, C3, ids, novl


def kernel(x, pos, batch, W1, b1, W2, b2):
    N, C = x.shape
    batch = batch.astype(jnp.int32)
    Q, C3, ids, novl = _prep_knn_inputs(pos, batch)
    idx = _knn_topk(Q, C3, ids, novl)

    x_sc = jnp.zeros((N_PAD, C), jnp.float32).at[:N].set(x)
    zeros_slab = jnp.zeros((N_PAD // 16, C), jnp.float32)
    parts = _scatter_variance(x_sc, idx.reshape(-1), zeros_slab)

    W1t = W1.T
    b1r = b1.reshape(1, 64)
    W2r = W2.reshape(1, 64)
    b2r = b2.reshape(1, 1)
    out = _mlp(parts[0], parts[1], W1t, b1r, W2r, b2r)
    return out[:N]


# speedup vs baseline: 7.2055x; 7.2055x over previous
import functools

import jax
import jax.numpy as jnp
from jax import lax
from jax.experimental import pallas as pl
from jax.experimental.pallas import tpu as pltpu
from jax.experimental.pallas import tpu_sc as plsc

K = 16
N_PAD = 10240
ROWS = 256
CHUNK = 512
N_TILES = N_PAD // ROWS
N_CHUNKS = N_PAD // CHUNK
BIG_I = 2 ** 30
F32_INF = float('inf')


def _knn_body(ids_ref, novl_ref, q_ref, c3_ref, idx_out_ref, d2_ref):
    t = pl.program_id(0)
    nov = novl_ref[t]
    qx = q_ref[:, 0:1]
    qy = q_ref[:, 1:2]
    qz = q_ref[:, 2:3]
    qb = q_ref[:, 3:4]

    def fill(k, carry):
        cid = ids_ref[t, k]
        cx = c3_ref[cid, 0:1, :]
        cy = c3_ref[cid, 1:2, :]
        cz = c3_ref[cid, 2:3, :]
        cb = c3_ref[cid, 3:4, :]
        dx = qx - cx
        dy = qy - cy
        dz = qz - cz
        d2 = dx * dx + dy * dy + dz * dz
        d2_ref[cid] = jnp.where(qb == cb, d2, F32_INF)
        return carry

    lax.fori_loop(0, nov, fill, 0)

    for tsel in range(K):
        def sel(k, mi):
            m, i = mi
            cid = ids_ref[t, k]
            d2c = d2_ref[cid]
            mc = jnp.min(d2c, axis=1, keepdims=True)
            colidx = (lax.broadcasted_iota(jnp.int32, (ROWS, CHUNK), 1)
                      + cid * CHUNK)
            ic = jnp.min(jnp.where(d2c == mc, colidx, BIG_I), axis=1,
                         keepdims=True)
            better = (mc < m) | ((mc == m) & (ic < i))
            return (jnp.where(better, mc, m), jnp.where(better, ic, i))

        m0 = jnp.full((ROWS, 1), F32_INF, jnp.float32)
        i0 = jnp.full((ROWS, 1), BIG_I, jnp.int32)
        m, i = lax.fori_loop(0, nov, sel, (m0, i0))

        def knock(k, carry):
            cid = ids_ref[t, k]
            d2c = d2_ref[cid]
            colidx = (lax.broadcasted_iota(jnp.int32, (ROWS, CHUNK), 1)
                      + cid * CHUNK)
            d2_ref[cid] = jnp.where((d2c == m) & (colidx == i), F32_INF, d2c)
            return carry

        lax.fori_loop(0, nov, knock, 0)
        safe = jnp.minimum(i, N_PAD - 1)
        idx_out_ref[:, tsel:tsel + 1] = jnp.where(qb == -2.0, N_PAD - 1, safe)


def _knn_topk(Q, C3, ids, novl, interpret=False):
    grid_spec = pltpu.PrefetchScalarGridSpec(
        num_scalar_prefetch=2,
        grid=(N_TILES,),
        in_specs=[
            pl.BlockSpec((ROWS, 4), lambda t, *_: (t, 0)),
            pl.BlockSpec((N_CHUNKS, 8, CHUNK), lambda t, *_: (0, 0, 0)),
        ],
        out_specs=pl.BlockSpec((ROWS, K), lambda t, *_: (t, 0)),
        scratch_shapes=[pltpu.VMEM((N_CHUNKS, ROWS, CHUNK), jnp.float32)],
    )
    return pl.pallas_call(
        _knn_body,
        grid_spec=grid_spec,
        out_shape=jax.ShapeDtypeStruct((N_PAD, K), jnp.int32),
        interpret=interpret,
    )(ids, novl, Q, C3)


def _scatter_variance(x_sc, idx_flat, zeros_slab):
    C = x_sc.shape[1]
    slab = N_PAD // 16
    qpt = N_PAD // 32
    n_blocks = qpt // 8
    mesh = plsc.VectorSubcoreMesh(core_axis_name="c", subcore_axis_name="s")

    @functools.partial(
        pl.kernel,
        mesh=mesh,
        out_type=jax.ShapeDtypeStruct((2, N_PAD, C), jnp.float32),
        scratch_types=[
            pltpu.VMEM((128,), jnp.int32),
            pltpu.VMEM((8, C), jnp.float32),
            pltpu.VMEM((128, C), jnp.float32),
            pltpu.VMEM_SHARED((N_PAD, C), jnp.float32),
            pltpu.SemaphoreType.DMA,
        ],
    )
    def sc_kernel(x_hbm, idxf_hbm, z_hbm, out_hbm, idx_v, cent_v, nbr_v,
                  var_sh, sem):
        c = lax.axis_index("c")
        s = lax.axis_index("s")
        base = (c * 16 + s) * qpt
        pltpu.sync_copy(z_hbm, var_sh.at[pl.ds(s * slab, slab)])
        plsc.subcore_barrier()

        def block(b, carry):
            q0 = base + b * 8
            pltpu.sync_copy(idxf_hbm.at[pl.ds(q0 * K, 128)], idx_v)
            pltpu.sync_copy(x_hbm.at[pl.ds(q0, 8)], cent_v)
            pltpu.async_copy(x_hbm.at[idx_v], nbr_v, sem).wait()

            def rowloop(r, carry2):
                q = lax.shift_right_logical(r, 4)
                for cc in range(C // 16):
                    sl = pl.ds(cc * 16, 16)
                    d = nbr_v[r, sl] - cent_v[q, sl]
                    nbr_v[r, sl] = d * d
                return carry2

            lax.fori_loop(0, 128, rowloop, 0)
            pltpu.sync_copy(nbr_v, var_sh.at[idx_v], add=True)
            return carry

        lax.fori_loop(0, n_blocks, block, 0)
        plsc.subcore_barrier()
        pltpu.sync_copy(var_sh.at[pl.ds(s * slab, slab)],
                        out_hbm.at[c, pl.ds(s * slab, slab)])

    return sc_kernel(x_sc, idx_flat, zeros_slab)


def _mlp_body(v0_ref, v1_ref, w1t_ref, b1_ref, w2_ref, b2_ref, out_ref):
    v = (v0_ref[...] + v1_ref[...]) * (1.0 / K)
    h = jnp.dot(v, w1t_ref[...], preferred_element_type=jnp.float32)
    h = jnp.maximum(h + b1_ref[...], 0.0)
    s = jnp.sum(h * w2_ref[...], axis=1, keepdims=True) + b2_ref[...]
    out_ref[...] = 1.0 / (1.0 + jnp.exp(-s))


def _mlp(var0, var1, W1t, b1r, W2r, b2r, interpret=False):
    RT = 512
    grid = (N_PAD // RT,)
    return pl.pallas_call(
        _mlp_body,
        grid=grid,
        in_specs=[
            pl.BlockSpec((RT, 128), lambda r: (r, 0)),
            pl.BlockSpec((RT, 128), lambda r: (r, 0)),
            pl.BlockSpec((128, 64), lambda r: (0, 0)),
            pl.BlockSpec((1, 64), lambda r: (0, 0)),
            pl.BlockSpec((1, 64), lambda r: (0, 0)),
            pl.BlockSpec((1, 1), lambda r: (0, 0)),
        ],
        out_specs=pl.BlockSpec((RT, 1), lambda r: (r, 0)),
        out_shape=jax.ShapeDtypeStruct((N_PAD, 1), jnp.float32),
        interpret=interpret,
    )(var0, var1, W1t, b1r, W2r, b2r)


def _prep_knn_inputs(pos, batch):
    N = pos.shape[0]
    bf = batch.astype(jnp.float32)
    Q = jnp.full((N_PAD, 4), -2.0, jnp.float32)
    Q = Q.at[:N, :3].set(pos)
    Q = Q.at[:N, 3].set(bf)
    Carr = jnp.zeros((8, N_PAD), jnp.float32)
    Carr = Carr.at[:3, :N].set(pos.T)
    Carr = Carr.at[3, :].set(-1.0)
    Carr = Carr.at[3, :N].set(bf)
    C3 = Carr.reshape(8, N_CHUNKS, CHUNK).transpose(1, 0, 2)
    tix = jnp.arange(N_TILES)
    cix = jnp.arange(N_CHUNKS)
    tmin = batch[jnp.minimum(tix * ROWS, N - 1)]
    tmax = batch[jnp.minimum((tix + 1) * ROWS - 1, N - 1)]
    cmin = batch[jnp.minimum(cix * CHUNK, N - 1)]
    cmax = batch[jnp.minimum((cix + 1) * CHUNK - 1, N - 1)]
    ovl = (cmax[None, :] >= tmin[:, None]) & (cmin[None, :] <= tmax[:, None])
    novl = jnp.sum(ovl.astype(jnp.int32), axis=1)
    ids = jnp.argsort(
        jnp.where(ovl, cix[None, :], N_CHUNKS + 100), axis=1
    ).astype(jnp.int32)
    return Q, C3, ids, novl


def kernel(x, pos, batch, W1, b1, W2, b2):
    N, C = x.shape
    batch = batch.astype(jnp.int32)
    Q, C3, ids, novl = _prep_knn_inputs(pos, batch)
    idx = _knn_topk(Q, C3, ids, novl)

    x_sc = jnp.zeros((N_PAD, C), jnp.float32).at[:N].set(x)
    zeros_slab = jnp.zeros((N_PAD // 16, C), jnp.float32)
    parts = _scatter_variance(x_sc, idx.reshape(-1), zeros_slab)

    W1t = W1.T
    b1r = b1.reshape(1, 64)
    W2r = W2.reshape(1, 64)
    b2r = b2.reshape(1, 1)
    out = _mlp(parts[0], parts[1], W1t, b1r, W2r, b2r)
    return out[:N]


# speedup vs baseline: 9.5681x; 1.3279x over previous
import functools

import jax
import jax.numpy as jnp
from jax import lax
from jax.experimental import pallas as pl
from jax.experimental.pallas import tpu as pltpu
from jax.experimental.pallas import tpu_sc as plsc

K = 16
N_PAD = 10240
ROWS = 256
CHUNK = 512
N_TILES = N_PAD // ROWS
N_CHUNKS = N_PAD // CHUNK
BIG_I = 2 ** 30
F32_INF = float('inf')


def _knn_body(t0, ids_ref, novl_ref, q_ref, c3_ref, idx_out_ref, d2_ref):
    t = pl.program_id(0) + t0
    nov = novl_ref[t]
    qx = q_ref[:, 0:1]
    qy = q_ref[:, 1:2]
    qz = q_ref[:, 2:3]
    qb = q_ref[:, 3:4]

    def fill(k, carry):
        cid = ids_ref[t, k]
        cx = c3_ref[cid, 0:1, :]
        cy = c3_ref[cid, 1:2, :]
        cz = c3_ref[cid, 2:3, :]
        cb = c3_ref[cid, 3:4, :]
        dx = qx - cx
        dy = qy - cy
        dz = qz - cz
        d2 = dx * dx + dy * dy + dz * dz
        d2_ref[cid] = jnp.where(qb == cb, d2, F32_INF)
        return carry

    lax.fori_loop(0, nov, fill, 0)

    pm = jnp.full((ROWS, 1), -F32_INF, jnp.float32)
    pi = jnp.full((ROWS, 1), -1, jnp.int32)
    for tsel in range(K):
        def sel(k, mi):
            m, i = mi
            cid = ids_ref[t, k]
            d2c = d2_ref[cid]
            colidx = (lax.broadcasted_iota(jnp.int32, (1, CHUNK), 1)
                      + cid * CHUNK)
            elig = (d2c > pm) | ((d2c == pm) & (colidx > pi))
            d2e = jnp.where(elig, d2c, F32_INF)
            mc = jnp.min(d2e, axis=1, keepdims=True)
            ic = jnp.min(jnp.where(d2e == mc, colidx, BIG_I), axis=1,
                         keepdims=True)
            better = (mc < m) | ((mc == m) & (ic < i))
            return (jnp.where(better, mc, m), jnp.where(better, ic, i))

        m0 = jnp.full((ROWS, 1), F32_INF, jnp.float32)
        i0 = jnp.full((ROWS, 1), BIG_I, jnp.int32)
        pm, pi = lax.fori_loop(0, nov, sel, (m0, i0))
        safe = jnp.minimum(pi, N_PAD - 1)
        idx_out_ref[:, tsel:tsel + 1] = jnp.where(qb == -2.0, N_PAD - 1, safe)


def _knn_topk(Q, C3, ids, novl, t0, nt, interpret=False):
    grid_spec = pltpu.PrefetchScalarGridSpec(
        num_scalar_prefetch=2,
        grid=(nt,),
        in_specs=[
            pl.BlockSpec((ROWS, 4), lambda t, *_: (t + t0, 0)),
            pl.BlockSpec((N_CHUNKS, 8, CHUNK), lambda t, *_: (0, 0, 0)),
        ],
        out_specs=pl.BlockSpec((ROWS, K), lambda t, *_: (t, 0)),
        scratch_shapes=[pltpu.VMEM((N_CHUNKS, ROWS, CHUNK), jnp.float32)],
    )
    return pl.pallas_call(
        functools.partial(_knn_body, t0),
        grid_spec=grid_spec,
        out_shape=jax.ShapeDtypeStruct((nt * ROWS, K), jnp.int32),
        interpret=interpret,
    )(ids, novl, Q, C3)


def _scatter_variance(x_sc, idx_flat, zeros_slab, q0_base, nq):
    C = x_sc.shape[1]
    slab = N_PAD // 16
    qpt = nq // 32
    n_blocks = qpt // 8
    mesh = plsc.VectorSubcoreMesh(core_axis_name="c", subcore_axis_name="s")

    @functools.partial(
        pl.kernel,
        mesh=mesh,
        out_type=jax.ShapeDtypeStruct((2, N_PAD, C), jnp.float32),
        scratch_types=[
            pltpu.VMEM((128,), jnp.int32),
            pltpu.VMEM((8, C), jnp.float32),
            pltpu.VMEM((128, C), jnp.float32),
            pltpu.VMEM_SHARED((N_PAD, C), jnp.float32),
            pltpu.SemaphoreType.DMA,
        ],
    )
    def sc_kernel(x_hbm, idxf_hbm, z_hbm, out_hbm, idx_v, cent_v, nbr_v,
                  var_sh, sem):
        c = lax.axis_index("c")
        s = lax.axis_index("s")
        base = (c * 16 + s) * qpt
        pltpu.sync_copy(z_hbm, var_sh.at[pl.ds(s * slab, slab)])
        plsc.subcore_barrier()

        def block(b, carry):
            q0 = base + b * 8
            pltpu.sync_copy(idxf_hbm.at[pl.ds(q0 * K, 128)], idx_v)
            pltpu.sync_copy(x_hbm.at[pl.ds(q0 + q0_base, 8)], cent_v)
            pltpu.async_copy(x_hbm.at[idx_v], nbr_v, sem).wait()

            def rowloop(r, carry2):
                q = lax.shift_right_logical(r, 4)
                for cc in range(C // 16):
                    sl = pl.ds(cc * 16, 16)
                    d = nbr_v[r, sl] - cent_v[q, sl]
                    nbr_v[r, sl] = d * d
                return carry2

            lax.fori_loop(0, 128, rowloop, 0)
            pltpu.sync_copy(nbr_v, var_sh.at[idx_v], add=True)
            return carry

        lax.fori_loop(0, n_blocks, block, 0)
        plsc.subcore_barrier()
        pltpu.sync_copy(var_sh.at[pl.ds(s * slab, slab)],
                        out_hbm.at[c, pl.ds(s * slab, slab)])

    return sc_kernel(x_sc, idx_flat, zeros_slab)


def _mlp_body(nparts, *refs):
    parts = refs[:nparts]
    w1t_ref, b1_ref, w2_ref, b2_ref, out_ref = refs[nparts:]
    v = parts[0][0] + parts[0][1]
    for pr in parts[1:]:
        v = v + pr[0] + pr[1]
    v = v * (1.0 / K)
    h = jnp.dot(v, w1t_ref[...], preferred_element_type=jnp.float32)
    h = jnp.maximum(h + b1_ref[...], 0.0)
    s = jnp.sum(h * w2_ref[...], axis=1, keepdims=True) + b2_ref[...]
    out_ref[...] = 1.0 / (1.0 + jnp.exp(-s))


def _mlp(parts_list, W1t, b1r, W2r, b2r, interpret=False):
    RT = 512
    grid = (N_PAD // RT,)
    part_specs = [pl.BlockSpec((2, RT, 128), lambda r: (0, r, 0))
                  for _ in parts_list]
    return pl.pallas_call(
        functools.partial(_mlp_body, len(parts_list)),
        grid=grid,
        in_specs=part_specs + [
            pl.BlockSpec((128, 64), lambda r: (0, 0)),
            pl.BlockSpec((1, 64), lambda r: (0, 0)),
            pl.BlockSpec((1, 64), lambda r: (0, 0)),
            pl.BlockSpec((1, 1), lambda r: (0, 0)),
        ],
        out_specs=pl.BlockSpec((RT, 1), lambda r: (r, 0)),
        out_shape=jax.ShapeDtypeStruct((N_PAD, 1), jnp.float32),
        interpret=interpret,
    )(*parts_list, W1t, b1r, W2r, b2r)


def _prep_knn_inputs(pos, batch):
    N = pos.shape[0]
    bf = batch.astype(jnp.float32)
    Q = jnp.full((N_PAD, 4), -2.0, jnp.float32)
    Q = Q.at[:N, :3].set(pos)
    Q = Q.at[:N, 3].set(bf)
    Carr = jnp.zeros((8, N_PAD), jnp.float32)
    Carr = Carr.at[:3, :N].set(pos.T)
    Carr = Carr.at[3, :].set(-1.0)
    Carr = Carr.at[3, :N].set(bf)
    C3 = Carr.reshape(8, N_CHUNKS, CHUNK).transpose(1, 0, 2)
    tix = jnp.arange(N_TILES)
    cix = jnp.arange(N_CHUNKS)
    tmin = batch[jnp.minimum(tix * ROWS, N - 1)]
    tmax = batch[jnp.minimum((tix + 1) * ROWS - 1, N - 1)]
    cmin = batch[jnp.minimum(cix * CHUNK, N - 1)]
    cmax = batch[jnp.minimum((cix + 1) * CHUNK - 1, N - 1)]
    ovl = (cmax[None, :] >= tmin[:, None]) & (cmin[None, :] <= tmax[:, None])
    novl = jnp.sum(ovl.astype(jnp.int32), axis=1)
    ids = jnp.argsort(
        jnp.where(ovl, cix[None, :], N_CHUNKS + 100), axis=1
    ).astype(jnp.int32)
    return Q, C3, ids, novl


def kernel(x, pos, batch, W1, b1, W2, b2):
    N, C = x.shape
    batch = batch.astype(jnp.int32)
    Q, C3, ids, novl = _prep_knn_inputs(pos, batch)
    x_sc = jnp.zeros((N_PAD, C), jnp.float32).at[:N].set(x)
    zeros_slab = jnp.zeros((N_PAD // 16, C), jnp.float32)

    NSPLIT = 4
    nt = N_TILES // NSPLIT
    nq = N_PAD // NSPLIT
    parts_list = []
    for kk in range(NSPLIT):
        idx_k = _knn_topk(Q, C3, ids, novl, kk * nt, nt)
        parts_list.append(_scatter_variance(x_sc, idx_k.reshape(-1),
                                            zeros_slab, kk * nq, nq))

    W1t = W1.T
    b1r = b1.reshape(1, 64)
    W2r = W2.reshape(1, 64)
    b2r = b2.reshape(1, 1)
    out = _mlp(parts_list, W1t, b1r, W2r, b2r)
    return out[:N]


# speedup vs baseline: 9.6381x; 1.0073x over previous
import functools

import jax
import jax.numpy as jnp
from jax import lax
from jax.experimental import pallas as pl
from jax.experimental.pallas import tpu as pltpu
from jax.experimental.pallas import tpu_sc as plsc

K = 16
N_PAD = 10240
ROWS = 256
CHUNK = 512
N_TILES = N_PAD // ROWS
N_CHUNKS = N_PAD // CHUNK
BIG_I = 2 ** 30
F32_INF = float('inf')


def _knn_body(t0, ids_ref, novl_ref, q_ref, c3_ref, idx_out_ref, d2_ref):
    t = pl.program_id(0) + t0
    nov = novl_ref[t]
    qx = q_ref[:, 0:1]
    qy = q_ref[:, 1:2]
    qz = q_ref[:, 2:3]
    qb = q_ref[:, 3:4]

    def fill(k, carry):
        cid = ids_ref[t, k]
        cx = c3_ref[cid, 0:1, :]
        cy = c3_ref[cid, 1:2, :]
        cz = c3_ref[cid, 2:3, :]
        cb = c3_ref[cid, 3:4, :]
        dx = qx - cx
        dy = qy - cy
        dz = qz - cz
        d2 = dx * dx + dy * dy + dz * dz
        d2_ref[cid] = jnp.where(qb == cb, d2, F32_INF)
        return carry

    lax.fori_loop(0, nov, fill, 0)

    pm = jnp.full((ROWS, 1), -F32_INF, jnp.float32)
    pi = jnp.full((ROWS, 1), -1, jnp.int32)
    for tsel in range(K):
        def sel(k, mi):
            m, i = mi
            cid = ids_ref[t, k]
            d2c = d2_ref[cid]
            colidx = (lax.broadcasted_iota(jnp.int32, (1, CHUNK), 1)
                      + cid * CHUNK)
            elig = (d2c > pm) | ((d2c == pm) & (colidx > pi))
            d2e = jnp.where(elig, d2c, F32_INF)
            mc = jnp.min(d2e, axis=1, keepdims=True)
            ic = jnp.min(jnp.where(d2e == mc, colidx, BIG_I), axis=1,
                         keepdims=True)
            better = (mc < m) | ((mc == m) & (ic < i))
            return (jnp.where(better, mc, m), jnp.where(better, ic, i))

        m0 = jnp.full((ROWS, 1), F32_INF, jnp.float32)
        i0 = jnp.full((ROWS, 1), BIG_I, jnp.int32)
        pm, pi = lax.fori_loop(0, nov, sel, (m0, i0))
        safe = jnp.minimum(pi, N_PAD - 1)
        idx_out_ref[:, tsel:tsel + 1] = jnp.where(qb == -2.0, N_PAD - 1, safe)


def _knn_topk(Q, C3, ids, novl, t0, nt, interpret=False):
    grid_spec = pltpu.PrefetchScalarGridSpec(
        num_scalar_prefetch=2,
        grid=(nt,),
        in_specs=[
            pl.BlockSpec((ROWS, 4), lambda t, *_: (t + t0, 0)),
            pl.BlockSpec((N_CHUNKS, 8, CHUNK), lambda t, *_: (0, 0, 0)),
        ],
        out_specs=pl.BlockSpec((ROWS, K), lambda t, *_: (t, 0)),
        scratch_shapes=[pltpu.VMEM((N_CHUNKS, ROWS, CHUNK), jnp.float32)],
    )
    return pl.pallas_call(
        functools.partial(_knn_body, t0),
        grid_spec=grid_spec,
        out_shape=jax.ShapeDtypeStruct((nt * ROWS, K), jnp.int32),
        interpret=interpret,
    )(ids, novl, Q, C3)


def _scatter_variance(x_sc, idx_flat, zeros_slab, q0_base, nq):
    C = x_sc.shape[1]
    slab = N_PAD // 16
    qpt = nq // 32
    n_blocks = qpt // 8
    mesh = plsc.VectorSubcoreMesh(core_axis_name="c", subcore_axis_name="s")

    @functools.partial(
        pl.kernel,
        mesh=mesh,
        out_type=jax.ShapeDtypeStruct((2, N_PAD, C), jnp.float32),
        scratch_types=[
            pltpu.VMEM((128,), jnp.int32),
            pltpu.VMEM((8, C), jnp.float32),
            pltpu.VMEM((128, C), jnp.float32),
            pltpu.VMEM_SHARED((N_PAD, C), jnp.float32),
            pltpu.SemaphoreType.DMA,
        ],
    )
    def sc_kernel(x_hbm, idxf_hbm, z_hbm, out_hbm, idx_v, cent_v, nbr_v,
                  var_sh, sem):
        c = lax.axis_index("c")
        s = lax.axis_index("s")
        base = (c * 16 + s) * qpt
        pltpu.sync_copy(z_hbm, var_sh.at[pl.ds(s * slab, slab)])
        plsc.subcore_barrier()

        def block(b, carry):
            q0 = base + b * 8
            pltpu.sync_copy(idxf_hbm.at[pl.ds(q0 * K, 128)], idx_v)
            pltpu.sync_copy(x_hbm.at[pl.ds(q0 + q0_base, 8)], cent_v)
            pltpu.async_copy(x_hbm.at[idx_v], nbr_v, sem).wait()

            def rowloop(r, carry2):
                q = lax.shift_right_logical(r, 4)
                for cc in range(C // 16):
                    sl = pl.ds(cc * 16, 16)
                    d = nbr_v[r, sl] - cent_v[q, sl]
                    nbr_v[r, sl] = d * d
                return carry2

            lax.fori_loop(0, 128, rowloop, 0)
            pltpu.sync_copy(nbr_v, var_sh.at[idx_v], add=True)
            return carry

        lax.fori_loop(0, n_blocks, block, 0)
        plsc.subcore_barrier()
        pltpu.sync_copy(var_sh.at[pl.ds(s * slab, slab)],
                        out_hbm.at[c, pl.ds(s * slab, slab)])

    return sc_kernel(x_sc, idx_flat, zeros_slab)


def _mlp_body(nparts, *refs):
    parts = refs[:nparts]
    w1t_ref, b1_ref, w2_ref, b2_ref, out_ref = refs[nparts:]
    v = parts[0][0] + parts[0][1]
    for pr in parts[1:]:
        v = v + pr[0] + pr[1]
    v = v * (1.0 / K)
    h = jnp.dot(v, w1t_ref[...], preferred_element_type=jnp.float32)
    h = jnp.maximum(h + b1_ref[...], 0.0)
    s = jnp.sum(h * w2_ref[...], axis=1, keepdims=True) + b2_ref[...]
    out_ref[...] = 1.0 / (1.0 + jnp.exp(-s))


def _mlp(parts_list, W1t, b1r, W2r, b2r, interpret=False):
    RT = 512
    grid = (N_PAD // RT,)
    part_specs = [pl.BlockSpec((2, RT, 128), lambda r: (0, r, 0))
                  for _ in parts_list]
    return pl.pallas_call(
        functools.partial(_mlp_body, len(parts_list)),
        grid=grid,
        in_specs=part_specs + [
            pl.BlockSpec((128, 64), lambda r: (0, 0)),
            pl.BlockSpec((1, 64), lambda r: (0, 0)),
            pl.BlockSpec((1, 64), lambda r: (0, 0)),
            pl.BlockSpec((1, 1), lambda r: (0, 0)),
        ],
        out_specs=pl.BlockSpec((RT, 1), lambda r: (r, 0)),
        out_shape=jax.ShapeDtypeStruct((N_PAD, 1), jnp.float32),
        interpret=interpret,
    )(*parts_list, W1t, b1r, W2r, b2r)


def _prep_knn_inputs(pos, batch):
    N = pos.shape[0]
    bf = batch.astype(jnp.float32)
    Q = jnp.full((N_PAD, 4), -2.0, jnp.float32)
    Q = Q.at[:N, :3].set(pos)
    Q = Q.at[:N, 3].set(bf)
    Carr = jnp.zeros((8, N_PAD), jnp.float32)
    Carr = Carr.at[:3, :N].set(pos.T)
    Carr = Carr.at[3, :].set(-1.0)
    Carr = Carr.at[3, :N].set(bf)
    C3 = Carr.reshape(8, N_CHUNKS, CHUNK).transpose(1, 0, 2)
    tix = jnp.arange(N_TILES)
    cix = jnp.arange(N_CHUNKS)
    tmin = batch[jnp.minimum(tix * ROWS, N - 1)]
    tmax = batch[jnp.minimum((tix + 1) * ROWS - 1, N - 1)]
    cmin = batch[jnp.minimum(cix * CHUNK, N - 1)]
    cmax = batch[jnp.minimum((cix + 1) * CHUNK - 1, N - 1)]
    ovl = (cmax[None, :] >= tmin[:, None]) & (cmin[None, :] <= tmax[:, None])
    novl = jnp.sum(ovl.astype(jnp.int32), axis=1)
    ids = jnp.argsort(
        jnp.where(ovl, cix[None, :], N_CHUNKS + 100), axis=1
    ).astype(jnp.int32)
    return Q, C3, ids, novl


def kernel(x, pos, batch, W1, b1, W2, b2):
    N, C = x.shape
    batch = batch.astype(jnp.int32)
    Q, C3, ids, novl = _prep_knn_inputs(pos, batch)
    x_sc = jnp.zeros((N_PAD, C), jnp.float32).at[:N].set(x)
    zeros_slab = jnp.zeros((N_PAD // 16, C), jnp.float32)

    NSPLIT = 8
    nt = N_TILES // NSPLIT
    nq = N_PAD // NSPLIT
    parts_list = []
    for kk in range(NSPLIT):
        idx_k = _knn_topk(Q, C3, ids, novl, kk * nt, nt)
        parts_list.append(_scatter_variance(x_sc, idx_k.reshape(-1),
                                            zeros_slab, kk * nq, nq))

    W1t = W1.T
    b1r = b1.reshape(1, 64)
    W2r = W2.reshape(1, 64)
    b2r = b2.reshape(1, 1)
    out = _mlp(parts_list, W1t, b1r, W2r, b2r)
    return out[:N]
